# baseline (speedup 1.0000x reference)
import functools

import jax
import jax.numpy as jnp
from jax import lax
from jax.experimental import pallas as pl
from jax.experimental.pallas import tpu as pltpu
from jax.experimental.pallas import tpu_sc as plsc

E = 320000
D = 128
NCH = 4
C5, S5 = 20000, 5
C6, S6 = 20000, 6
N5 = C5 * S5
N6 = C6 * S6

WG = 512
P5 = 114688
P6 = 131072

RW = 12160
NRANGE = 27
EPAD = NRANGE * RW
ACC_ROWS = 12288
WT = 128
WIN = 16 * WT
NSTREAM = 80
NWINTOT = 164
NPAD = NWINTOT * WIN

BC5, BR5 = 128, 128 * S5
BC6, BR6 = 128, 128 * S6
BE = 1024


def _gather_sc(table, idx0, idx1, P):
    per_w = P // 32
    nwin = per_w // WG
    mesh = plsc.VectorSubcoreMesh(core_axis_name="c", subcore_axis_name="s")

    @functools.partial(
        pl.kernel, mesh=mesh,
        out_type=(jax.ShapeDtypeStruct((P, D), jnp.float32),
                  jax.ShapeDtypeStruct((P, D), jnp.float32)),
        scratch_types=[
            pltpu.VMEM((WG,), jnp.int32),
            pltpu.VMEM((WG, D), jnp.float32),
            pltpu.SemaphoreType.DMA,
        ],
    )
    def k(tab, i0, i1, g0, g1, idx_v, rows_v, sem):
        cid = lax.axis_index("c")
        sid = lax.axis_index("s")
        wid = sid * 2 + cid
        base = wid * per_w
        for src, dst in ((i0, g0), (i1, g1)):
            for j in range(nwin):
                off = base + j * WG
                pltpu.sync_copy(src.at[pl.ds(off, WG)], idx_v)
                pltpu.async_copy(tab.at[idx_v], rows_v, sem).wait()
                pltpu.sync_copy(rows_v, dst.at[pl.ds(off, WG)])

    return k(table, idx0, idx1)


def _scatter_sc(o5, o6, sp, slp, nwin_t, woff_t, zz):
    mesh = plsc.VectorSubcoreMesh(core_axis_name="c", subcore_axis_name="s")

    @functools.partial(
        pl.kernel, mesh=mesh,
        out_type=jax.ShapeDtypeStruct((EPAD, D), jnp.float32),
        scratch_types=[
            pltpu.VMEM_SHARED((ACC_ROWS, D), jnp.float32),
            pltpu.VMEM((64, D), jnp.float32),
            pltpu.VMEM((WT,), jnp.int32),
            pltpu.VMEM((WT,), jnp.int32),
            pltpu.VMEM((WT, D), jnp.float32),
            pltpu.VMEM((NSTREAM,), jnp.int32),
            pltpu.VMEM((NSTREAM,), jnp.int32),
            pltpu.SemaphoreType.DMA,
        ],
    )
    def k(out5_h, out6_h, sp_h, slp_h, nwin_h, woff_h, zz_h, out,
          acc, zbuf, lidx_v, gidx_v, rows_v, nwin_v, woff_v, sem):
        cid = lax.axis_index("c")
        tid = lax.axis_index("s")
        pltpu.sync_copy(zz_h, zbuf)
        pltpu.sync_copy(nwin_h, nwin_v)
        pltpu.sync_copy(woff_h, woff_v)

        def body(rr, carry):
            r = 2 * rr + cid

            @pl.when(r < NRANGE)
            def _():
                for kk in range(12):
                    pltpu.sync_copy(
                        zbuf, acc.at[pl.ds((tid * 12 + kk) * 64, 64)])
                plsc.subcore_barrier()
                for s, src in ((0, out5_h), (1, out6_h)):
                    sid_ = 2 * r + s
                    nw = nwin_v[pl.ds(sid_, 16)][0]
                    wo = woff_v[pl.ds(sid_, 16)][0]

                    def wbody(w, c2):
                        off = (wo + w) * WIN + tid * WT
                        pltpu.sync_copy(slp_h.at[pl.ds(off, WT)], lidx_v)
                        pltpu.sync_copy(sp_h.at[pl.ds(off, WT)], gidx_v)
                        pltpu.async_copy(src.at[gidx_v], rows_v, sem).wait()
                        pltpu.sync_copy(rows_v, acc.at[lidx_v], add=True)
                        return c2

                    lax.fori_loop(0, nw, wbody, 0)
                plsc.subcore_barrier()
                pltpu.sync_copy(acc.at[pl.ds(tid * 760, 760)],
                                out.at[pl.ds(r * RW + tid * 760, 760)])
                plsc.subcore_barrier()

            return carry

        lax.fori_loop(0, 14, body, 0)

    return k(o5, o6, sp, slp, nwin_t, woff_t, zz)


def _branch_body(n_rows, br, cyc_r, g0_r, g1_r, w1a, w1b, b1, w2, b2, wx, wp,
                 b1p, cw2, cb2, cw3, cb3, amat, out_r):
    e2c = g0_r[...] + g1_r[...]
    f32 = jnp.float32
    h = jnp.maximum(
        jnp.dot(cyc_r[...], w1a[...], preferred_element_type=f32)
        + jnp.dot(e2c, w1b[...], preferred_element_type=f32) + b1[...], 0.0)
    new = jnp.maximum(
        jnp.dot(h, w2[...], preferred_element_type=f32) + b2[...], 0.0)
    row = pl.program_id(0) * br + lax.broadcasted_iota(jnp.int32, (br, 1), 0)
    new = jnp.where(row < n_rows, new, 0.0)
    sums = jnp.dot(amat[...], new, preferred_element_type=f32)
    t = jnp.dot(sums, wp[...], preferred_element_type=f32)
    u = lax.dot_general(amat[...], t, (((0,), (0,)), ((), ())),
                        preferred_element_type=f32)
    h1 = jnp.maximum(
        jnp.dot(new, wx[...], preferred_element_type=f32) + u + b1p[...], 0.0)
    h2 = jnp.maximum(
        jnp.dot(h1, cw2[...], preferred_element_type=f32) + cb2[...], 0.0)
    out_r[...] = jnp.maximum(
        jnp.dot(h2, cw3[...], preferred_element_type=f32) + cb3[...], 0.0)


def _branch_call(cyc, g0, g1, n_rows, br, weights):
    nblk = (n_rows + br - 1) // br
    row_spec = pl.BlockSpec((br, D), lambda i: (i, 0))

    def wspec(a):
        return pl.BlockSpec(a.shape, lambda i: tuple(0 for _ in a.shape))

    in_specs = [row_spec, row_spec, row_spec] + [wspec(a) for a in weights]
    return pl.pallas_call(
        functools.partial(_branch_body, n_rows, br),
        grid=(nblk,),
        in_specs=in_specs,
        out_specs=pl.BlockSpec((br, D), lambda i: (i, 0)),
        out_shape=jax.ShapeDtypeStruct((n_rows, D), jnp.float32),
    )(cyc, g0, g1, *weights)


def _edge_body(e_r, c_r, w1a, w1b, b1, w2, b2, out_r):
    f32 = jnp.float32
    h = jnp.maximum(
        jnp.dot(e_r[...], w1a[...], preferred_element_type=f32)
        + jnp.dot(c_r[...], w1b[...], preferred_element_type=f32) + b1[...],
        0.0)
    out_r[...] = jnp.maximum(
        jnp.dot(h, w2[...], preferred_element_type=f32) + b2[...], 0.0)


def _edge_call(edge_rep, c2e_pad, weights):
    nblk = (E + BE - 1) // BE
    row_spec = pl.BlockSpec((BE, D), lambda i: (i, 0))

    def wspec(a):
        return pl.BlockSpec(a.shape, lambda i: tuple(0 for _ in a.shape))

    in_specs = [row_spec, row_spec] + [wspec(a) for a in weights]
    return pl.pallas_call(
        _edge_body,
        grid=(nblk,),
        in_specs=in_specs,
        out_specs=pl.BlockSpec((BE, D), lambda i: (i, 0)),
        out_shape=jax.ShapeDtypeStruct((E, D), jnp.float32),
    )(edge_rep, c2e_pad, *weights)


def _pad_idx(col, P):
    n = col.shape[0]
    filler = (jnp.arange(P - n, dtype=jnp.int32) % 997)
    return jnp.concatenate([col.astype(jnp.int32), filler])


def _pool_mat(bc, s):
    rows = jnp.arange(bc * s, dtype=jnp.int32) // s
    return (rows[None, :] == jnp.arange(bc, dtype=jnp.int32)[:, None]).astype(
        jnp.float32)


def kernel(edge_rep, cycle5_rep, cycle6_rep, e2c5_idx, e2c6_idx, c2e5_idx,
           c2e6_idx, mlp2_w1, mlp2_b1, mlp2_w2, mlp2_b2, cmlp_w1, cmlp_b1,
           cmlp_w2, cmlp_b2, cmlp_w3, cmlp_b3, emlp_w1, emlp_b1, emlp_w2,
           emlp_b2, aut5_w1, aut5_w2, aut5_b, aut6_w1, aut6_w2, aut6_b):
    i32 = jnp.int32

    gi5a = _pad_idx(e2c5_idx[:, 0], P5)
    gi5b = _pad_idx(e2c5_idx[:, 1], P5)
    gi6a = _pad_idx(e2c6_idx[:, 0], P6)
    gi6b = _pad_idx(e2c6_idx[:, 1], P6)

    c2e = jnp.concatenate([c2e5_idx, c2e6_idx]).astype(i32)
    nt = N5 + N6
    srcrow = jnp.concatenate([jnp.arange(N5, dtype=i32),
                              jnp.arange(N6, dtype=i32)])
    rg = c2e // RW
    stream = rg * 2 + jnp.concatenate(
        [jnp.zeros((N5,), i32), jnp.ones((N6,), i32)])
    ordr = jnp.argsort(stream)
    sstream = stream[ordr]
    cnt = jnp.bincount(stream, length=NSTREAM).astype(i32)
    nwin = (cnt + WIN - 1) // WIN
    woff = jnp.concatenate([jnp.zeros((1,), i32), jnp.cumsum(nwin)[:-1]])
    ptr = jnp.concatenate([jnp.zeros((1,), i32), jnp.cumsum(cnt)[:-1]])
    kk = jnp.arange(nt, dtype=i32) - ptr[sstream]
    p = woff[sstream] * WIN + kk
    sp = (jnp.arange(NPAD, dtype=i32) % 65536).at[p].set(srcrow[ordr])
    slp = (RW + jnp.arange(NPAD, dtype=i32) % 128).at[p].set(
        (c2e - rg * RW)[ordr])
    zz = jnp.zeros((64, D), jnp.float32)

    w1x = cmlp_w1[:D]
    w1s = cmlp_w1[D:2 * D]
    w1a = cmlp_w1[2 * D:]
    wx5 = w1x + aut5_w1 @ w1a
    wp5 = w1s + (aut5_w2 @ w1a) / S5
    b1p5 = (cmlp_b1 + aut5_b @ w1a).reshape(1, -1)
    wx6 = w1x + aut6_w1 @ w1a
    wp6 = w1s + (aut6_w2 @ w1a) / S6
    b1p6 = (cmlp_b1 + aut6_b @ w1a).reshape(1, -1)

    m2w1a = mlp2_w1[:D]
    m2w1b = mlp2_w1[D:]
    m2b1 = mlp2_b1.reshape(1, -1)
    m2b2 = mlp2_b2.reshape(1, -1)
    cb2 = cmlp_b2.reshape(1, -1)
    cb3 = cmlp_b3.reshape(1, -1)
    ew1a = emlp_w1[:D]
    ew1b = emlp_w1[D:]
    eb1 = emlp_b1.reshape(1, -1)
    eb2 = emlp_b2.reshape(1, -1)

    g5a, g5b = _gather_sc(edge_rep, gi5a, gi5b, P5)
    g6a, g6b = _gather_sc(edge_rep, gi6a, gi6b, P6)

    w5 = (m2w1a, m2w1b, m2b1, mlp2_w2, m2b2, wx5, wp5, b1p5,
          cmlp_w2, cb2, cmlp_w3, cb3, _pool_mat(BC5, S5))
    w6 = (m2w1a, m2w1b, m2b1, mlp2_w2, m2b2, wx6, wp6, b1p6,
          cmlp_w2, cb2, cmlp_w3, cb3, _pool_mat(BC6, S6))
    out5 = _branch_call(cycle5_rep, g5a, g5b, N5, BR5, w5)
    out6 = _branch_call(cycle6_rep, g6a, g6b, N6, BR6, w6)

    c2e_pad = _scatter_sc(out5, out6, sp, slp, nwin, woff, zz)

    edge_out = _edge_call(edge_rep, c2e_pad, (ew1a, ew1b, eb1, emlp_w2, eb2))
    return (edge_out, out5, out6)


# speedup vs baseline: 3.9073x; 3.9073x over previous
import functools

import jax
import jax.numpy as jnp
from jax import lax
from jax.experimental import pallas as pl
from jax.experimental.pallas import tpu as pltpu
from jax.experimental.pallas import tpu_sc as plsc

E = 320000
D = 128
NCH = 4
C5, S5 = 20000, 5
C6, S6 = 20000, 6
N5 = C5 * S5
N6 = C6 * S6

WG = 512
P5 = 114688
P6 = 131072

RW = 12160
NRANGE = 27
EPAD = NRANGE * RW
ACC_ROWS = 12288
WT = 128
WIN = 16 * WT
NSTREAM = 80
NTOT = N5 + N6
NS_PAD = 223232

BC5, BR5 = 128, 128 * S5
BC6, BR6 = 128, 128 * S6
BE = 1024


def _gather_sc(table, idx0, idx1, P):
    per_w = P // 32
    nwin = per_w // WG
    mesh = plsc.VectorSubcoreMesh(core_axis_name="c", subcore_axis_name="s")

    @functools.partial(
        pl.kernel, mesh=mesh,
        out_type=(jax.ShapeDtypeStruct((P, D), jnp.float32),
                  jax.ShapeDtypeStruct((P, D), jnp.float32)),
        scratch_types=[
            pltpu.VMEM((WG,), jnp.int32),
            pltpu.VMEM((WG, D), jnp.float32),
            pltpu.SemaphoreType.DMA,
        ],
    )
    def k(tab, i0, i1, g0, g1, idx_v, rows_v, sem):
        cid = lax.axis_index("c")
        sid = lax.axis_index("s")
        wid = sid * 2 + cid
        base = wid * per_w
        for src, dst in ((i0, g0), (i1, g1)):
            for j in range(nwin):
                off = base + j * WG
                pltpu.sync_copy(src.at[pl.ds(off, WG)], idx_v)
                pltpu.async_copy(tab.at[idx_v], rows_v, sem).wait()
                pltpu.sync_copy(rows_v, dst.at[pl.ds(off, WG)])

    return k(table, idx0, idx1)


def _scatter_sc(o5, o6, ordp, rowsrc, lml, lo_t, hi_t, nwin_t, zz):
    mesh = plsc.VectorSubcoreMesh(core_axis_name="c", subcore_axis_name="s")

    @functools.partial(
        pl.kernel, mesh=mesh,
        out_type=jax.ShapeDtypeStruct((EPAD, D), jnp.float32),
        scratch_types=[
            pltpu.VMEM_SHARED((ACC_ROWS, D), jnp.float32),
            pltpu.VMEM((64, D), jnp.float32),
            pltpu.VMEM((WT,), jnp.int32),
            pltpu.VMEM((WT,), jnp.int32),
            pltpu.VMEM((WT,), jnp.int32),
            pltpu.VMEM((WT, D), jnp.float32),
            pltpu.VMEM((NSTREAM,), jnp.int32),
            pltpu.VMEM((NSTREAM,), jnp.int32),
            pltpu.VMEM((NSTREAM,), jnp.int32),
            pltpu.SemaphoreType.DMA,
        ],
    )
    def k(out5_h, out6_h, ord_h, rowsrc_h, lml_h, lo_h, hi_h, nwin_h, zz_h,
          out, acc, zbuf, ord_v, gidx_v, lidx_v, rows_v, lo_v, hi_v, nwin_v,
          sem):
        cid = lax.axis_index("c")
        tid = lax.axis_index("s")
        pltpu.sync_copy(zz_h, zbuf)
        pltpu.sync_copy(lo_h, lo_v)
        pltpu.sync_copy(hi_h, hi_v)
        pltpu.sync_copy(nwin_h, nwin_v)

        def body(rr, carry):
            r = 2 * rr + cid

            @pl.when(r < NRANGE)
            def _():
                for kk in range(12):
                    pltpu.sync_copy(
                        zbuf, acc.at[pl.ds((tid * 12 + kk) * 64, 64)])
                plsc.subcore_barrier()
                for s, src in ((0, out5_h), (1, out6_h)):
                    sid_ = 2 * r + s
                    nw = nwin_v[pl.ds(sid_, 16)][0]
                    lo = lo_v[pl.ds(sid_, 16)][0]
                    hi = hi_v[pl.ds(sid_, 16)][0]
                    base0 = (lo // WT) * WT

                    def wbody(w, c2):
                        base = base0 + w * WIN + tid * WT
                        pltpu.sync_copy(ord_h.at[pl.ds(base, WT)], ord_v)
                        pltpu.async_copy(
                            rowsrc_h.at[ord_v], gidx_v, sem).wait()
                        pltpu.async_copy(lml_h.at[ord_v], lidx_v, sem).wait()
                        for v in range(WT // 16):
                            pos = (base + v * 16
                                   + lax.iota(jnp.int32, 16))
                            ok = (pos >= lo) & (pos < hi)
                            lane = lax.iota(jnp.int32, 16) + v * 16
                            lv = lidx_v[pl.ds(v * 16, 16)]
                            gv = gidx_v[pl.ds(v * 16, 16)]
                            lidx_v[pl.ds(v * 16, 16)] = jnp.where(
                                ok, lv, RW + lane)
                            gidx_v[pl.ds(v * 16, 16)] = jnp.where(
                                ok, gv, lane)
                        pltpu.async_copy(src.at[gidx_v], rows_v, sem).wait()
                        pltpu.sync_copy(rows_v, acc.at[lidx_v], add=True)
                        return c2

                    lax.fori_loop(0, nw, wbody, 0)
                plsc.subcore_barrier()
                pltpu.sync_copy(acc.at[pl.ds(tid * 760, 760)],
                                out.at[pl.ds(r * RW + tid * 760, 760)])
                plsc.subcore_barrier()

            return carry

        lax.fori_loop(0, 14, body, 0)

    return k(o5, o6, ordp, rowsrc, lml, lo_t, hi_t, nwin_t, zz)


def _branch_body(n_rows, br, cyc_r, g0_r, g1_r, w1a, w1b, b1, w2, b2, wx, wp,
                 b1p, cw2, cb2, cw3, cb3, amat, out_r):
    e2c = g0_r[...] + g1_r[...]
    f32 = jnp.float32
    h = jnp.maximum(
        jnp.dot(cyc_r[...], w1a[...], preferred_element_type=f32)
        + jnp.dot(e2c, w1b[...], preferred_element_type=f32) + b1[...], 0.0)
    new = jnp.maximum(
        jnp.dot(h, w2[...], preferred_element_type=f32) + b2[...], 0.0)
    row = pl.program_id(0) * br + lax.broadcasted_iota(jnp.int32, (br, 1), 0)
    new = jnp.where(row < n_rows, new, 0.0)
    sums = jnp.dot(amat[...], new, preferred_element_type=f32)
    t = jnp.dot(sums, wp[...], preferred_element_type=f32)
    u = lax.dot_general(amat[...], t, (((0,), (0,)), ((), ())),
                        preferred_element_type=f32)
    h1 = jnp.maximum(
        jnp.dot(new, wx[...], preferred_element_type=f32) + u + b1p[...], 0.0)
    h2 = jnp.maximum(
        jnp.dot(h1, cw2[...], preferred_element_type=f32) + cb2[...], 0.0)
    out_r[...] = jnp.maximum(
        jnp.dot(h2, cw3[...], preferred_element_type=f32) + cb3[...], 0.0)


def _branch_call(cyc, g0, g1, n_rows, br, weights):
    nblk = (n_rows + br - 1) // br
    row_spec = pl.BlockSpec((br, D), lambda i: (i, 0))

    def wspec(a):
        return pl.BlockSpec(a.shape, lambda i: tuple(0 for _ in a.shape))

    in_specs = [row_spec, row_spec, row_spec] + [wspec(a) for a in weights]
    return pl.pallas_call(
        functools.partial(_branch_body, n_rows, br),
        grid=(nblk,),
        in_specs=in_specs,
        out_specs=pl.BlockSpec((br, D), lambda i: (i, 0)),
        out_shape=jax.ShapeDtypeStruct((n_rows, D), jnp.float32),
    )(cyc, g0, g1, *weights)


def _edge_body(e_r, c_r, w1a, w1b, b1, w2, b2, out_r):
    f32 = jnp.float32
    h = jnp.maximum(
        jnp.dot(e_r[...], w1a[...], preferred_element_type=f32)
        + jnp.dot(c_r[...], w1b[...], preferred_element_type=f32) + b1[...],
        0.0)
    out_r[...] = jnp.maximum(
        jnp.dot(h, w2[...], preferred_element_type=f32) + b2[...], 0.0)


def _edge_call(edge_rep, c2e_pad, weights):
    nblk = (E + BE - 1) // BE
    row_spec = pl.BlockSpec((BE, D), lambda i: (i, 0))

    def wspec(a):
        return pl.BlockSpec(a.shape, lambda i: tuple(0 for _ in a.shape))

    in_specs = [row_spec, row_spec] + [wspec(a) for a in weights]
    return pl.pallas_call(
        _edge_body,
        grid=(nblk,),
        in_specs=in_specs,
        out_specs=pl.BlockSpec((BE, D), lambda i: (i, 0)),
        out_shape=jax.ShapeDtypeStruct((E, D), jnp.float32),
    )(edge_rep, c2e_pad, *weights)


def _pad_idx(col, P):
    n = col.shape[0]
    filler = (jnp.arange(P - n, dtype=jnp.int32) % 997)
    return jnp.concatenate([col.astype(jnp.int32), filler])


def _pool_mat(bc, s):
    rows = jnp.arange(bc * s, dtype=jnp.int32) // s
    return (rows[None, :] == jnp.arange(bc, dtype=jnp.int32)[:, None]).astype(
        jnp.float32)


def kernel(edge_rep, cycle5_rep, cycle6_rep, e2c5_idx, e2c6_idx, c2e5_idx,
           c2e6_idx, mlp2_w1, mlp2_b1, mlp2_w2, mlp2_b2, cmlp_w1, cmlp_b1,
           cmlp_w2, cmlp_b2, cmlp_w3, cmlp_b3, emlp_w1, emlp_b1, emlp_w2,
           emlp_b2, aut5_w1, aut5_w2, aut5_b, aut6_w1, aut6_w2, aut6_b):
    i32 = jnp.int32

    gi5a = _pad_idx(e2c5_idx[:, 0], P5)
    gi5b = _pad_idx(e2c5_idx[:, 1], P5)
    gi6a = _pad_idx(e2c6_idx[:, 0], P6)
    gi6b = _pad_idx(e2c6_idx[:, 1], P6)

    c2e = jnp.concatenate([c2e5_idx, c2e6_idx]).astype(i32)
    pos = jnp.arange(NTOT, dtype=i32)
    rg = c2e // RW
    stream = rg * 2 + (pos >= N5).astype(i32)
    key = stream * 262144 + pos
    skey = jnp.sort(key)
    sstream = skey // 262144
    ordv = skey - sstream * 262144
    ordp = jnp.concatenate([ordv, jnp.zeros((NS_PAD - NTOT,), i32)])
    rowsrc = jnp.concatenate([jnp.arange(N5, dtype=i32),
                              jnp.arange(N6, dtype=i32)])
    lml = c2e - rg * RW
    onehot = sstream[:, None] == jnp.arange(NSTREAM, dtype=i32)[None, :]
    cnt = jnp.sum(onehot, axis=0, dtype=i32)
    hi = jnp.cumsum(cnt).astype(i32)
    lo = hi - cnt
    base0 = (lo // WT) * WT
    nwin = jnp.where(cnt > 0, (hi - base0 + WIN - 1) // WIN, 0)
    zz = jnp.zeros((64, D), jnp.float32)

    w1x = cmlp_w1[:D]
    w1s = cmlp_w1[D:2 * D]
    w1a = cmlp_w1[2 * D:]
    wx5 = w1x + aut5_w1 @ w1a
    wp5 = w1s + (aut5_w2 @ w1a) / S5
    b1p5 = (cmlp_b1 + aut5_b @ w1a).reshape(1, -1)
    wx6 = w1x + aut6_w1 @ w1a
    wp6 = w1s + (aut6_w2 @ w1a) / S6
    b1p6 = (cmlp_b1 + aut6_b @ w1a).reshape(1, -1)

    m2w1a = mlp2_w1[:D]
    m2w1b = mlp2_w1[D:]
    m2b1 = mlp2_b1.reshape(1, -1)
    m2b2 = mlp2_b2.reshape(1, -1)
    cb2 = cmlp_b2.reshape(1, -1)
    cb3 = cmlp_b3.reshape(1, -1)
    ew1a = emlp_w1[:D]
    ew1b = emlp_w1[D:]
    eb1 = emlp_b1.reshape(1, -1)
    eb2 = emlp_b2.reshape(1, -1)

    g5a, g5b = _gather_sc(edge_rep, gi5a, gi5b, P5)
    g6a, g6b = _gather_sc(edge_rep, gi6a, gi6b, P6)

    w5 = (m2w1a, m2w1b, m2b1, mlp2_w2, m2b2, wx5, wp5, b1p5,
          cmlp_w2, cb2, cmlp_w3, cb3, _pool_mat(BC5, S5))
    w6 = (m2w1a, m2w1b, m2b1, mlp2_w2, m2b2, wx6, wp6, b1p6,
          cmlp_w2, cb2, cmlp_w3, cb3, _pool_mat(BC6, S6))
    out5 = _branch_call(cycle5_rep, g5a, g5b, N5, BR5, w5)
    out6 = _branch_call(cycle6_rep, g6a, g6b, N6, BR6, w6)

    c2e_pad = _scatter_sc(out5, out6, ordp, rowsrc, lml, lo, hi, nwin, zz)

    edge_out = _edge_call(edge_rep, c2e_pad, (ew1a, ew1b, eb1, emlp_w2, eb2))
    return (edge_out, out5, out6)
